# baseline (speedup 1.0000x reference)
import functools

import jax
import jax.numpy as jnp
from jax import lax
from jax.experimental import pallas as pl
from jax.experimental.pallas import tpu as pltpu
from jax.experimental.pallas import tpu_sc as plsc

_NUM_LAYERS = 16
_HIDDEN = 768
_INNER = 1536
_N_TOKENS = 32768

_TM = 256
_NTILES = _N_TOKENS // _TM
_NSTEP = _NTILES + _NUM_LAYERS - 1


_NC = 2
_NS = 16
_NW = _NC * _NS


def _make_sc_gather(n_rows, d):
    rpw = n_rows // _NW
    chunk = 64
    nch = rpw // chunk
    mesh = plsc.VectorSubcoreMesh(core_axis_name="c", subcore_axis_name="s")

    @functools.partial(
        pl.kernel,
        mesh=mesh,
        out_type=jax.ShapeDtypeStruct((n_rows, d), jnp.float32),
        scratch_types=[
            pltpu.VMEM((rpw,), jnp.int32),
            pltpu.VMEM((2, chunk, d), jnp.float32),
            pltpu.SemaphoreType.DMA,
            pltpu.SemaphoreType.DMA,
            pltpu.SemaphoreType.DMA,
            pltpu.SemaphoreType.DMA,
        ],
    )
    def gather_k(src_hbm, idx_hbm, out_hbm, idx_v, buf_v, gsem0, gsem1,
                 wsem0, wsem1):
        cid = lax.axis_index("c")
        sid = lax.axis_index("s")
        wid = sid * _NC + cid
        base = wid * rpw
        pltpu.sync_copy(idx_hbm.at[pl.ds(base, rpw)], idx_v)
        gsem = (gsem0, gsem1)
        wsem = (wsem0, wsem1)
        g_cp = [None, None]
        w_cp = [None, None]
        for c in range(nch):
            b = c % 2
            if w_cp[b] is not None:
                w_cp[b].wait()
            g_cp[b] = pltpu.async_copy(
                src_hbm.at[idx_v.at[pl.ds(c * chunk, chunk)]],
                buf_v.at[b], gsem[b])
            if c >= 1:
                pb = (c - 1) % 2
                g_cp[pb].wait()
                w_cp[pb] = pltpu.async_copy(
                    buf_v.at[pb],
                    out_hbm.at[pl.ds(base + (c - 1) * chunk, chunk)],
                    wsem[pb])
        lb = (nch - 1) % 2
        g_cp[lb].wait()
        w_cp[lb] = pltpu.async_copy(
            buf_v.at[lb],
            out_hbm.at[pl.ds(base + (nch - 1) * chunk, chunk)], wsem[lb])
        w_cp[(nch - 2) % 2].wait()
        w_cp[lb].wait()

    return gather_k


_make_sc_gather = functools.lru_cache(maxsize=None)(_make_sc_gather)


def _gelu(x):
    return 0.5 * x * (1.0 + lax.erf(x * (2.0 ** -0.5)))


def _gmm_body(st_ref, sg_ref, starts_ref, ends_ref,
              zs_ref, w1_ref, b1_ref, w2_ref, b2_ref, out_ref):
    i = pl.program_id(0)
    t = st_ref[i]
    g = sg_ref[i]
    rows = t * _TM + lax.broadcasted_iota(jnp.int32, (_TM, 1), 0)
    mask = (rows >= starts_ref[g]) & (rows < ends_ref[g])
    x = zs_ref[...]
    h = lax.dot_general(x.astype(jnp.bfloat16), w1_ref[0],
                        (((1,), (1,)), ((), ())),
                        preferred_element_type=jnp.float32)
    h = _gelu(h + b1_ref[0])
    y = lax.dot_general(h.astype(jnp.bfloat16), w2_ref[0],
                        (((1,), (1,)), ((), ())),
                        preferred_element_type=jnp.float32)
    y = y + b2_ref[0] + x
    out_ref[...] = jnp.where(mask, y, out_ref[...])


def _gmm(zs, W1, b1, W2, b2, step_t, step_g, starts, ends):
    grid_spec = pltpu.PrefetchScalarGridSpec(
        num_scalar_prefetch=4,
        grid=(_NSTEP,),
        in_specs=[
            pl.BlockSpec((_TM, _HIDDEN), lambda i, st, sg, s0, e0: (st[i], 0)),
            pl.BlockSpec((1, _INNER, _HIDDEN),
                         lambda i, st, sg, s0, e0: (sg[i], 0, 0)),
            pl.BlockSpec((1, 1, _INNER), lambda i, st, sg, s0, e0: (sg[i], 0, 0)),
            pl.BlockSpec((1, _HIDDEN, _INNER),
                         lambda i, st, sg, s0, e0: (sg[i], 0, 0)),
            pl.BlockSpec((1, 1, _HIDDEN), lambda i, st, sg, s0, e0: (sg[i], 0, 0)),
        ],
        out_specs=pl.BlockSpec((_TM, _HIDDEN),
                               lambda i, st, sg, s0, e0: (st[i], 0)),
    )
    return pl.pallas_call(
        _gmm_body,
        grid_spec=grid_spec,
        out_shape=jax.ShapeDtypeStruct((_N_TOKENS, _HIDDEN), jnp.float32),
        compiler_params=pltpu.CompilerParams(
            dimension_semantics=("arbitrary",)),
    )(step_t, step_g, starts, ends, zs, W1.astype(jnp.bfloat16),
      b1.reshape(_NUM_LAYERS, 1, _INNER), W2.astype(jnp.bfloat16),
      b2.reshape(_NUM_LAYERS, 1, _HIDDEN))


def _schedule(ids):
    sizes = jnp.bincount(ids, length=_NUM_LAYERS)
    ends = jnp.cumsum(sizes)
    starts = ends - sizes
    first = starts // _TM
    count = jnp.where(sizes > 0, (ends - 1) // _TM - first + 1, 0)
    csum = jnp.cumsum(count)
    base = csum - count
    sidx = jnp.arange(_NSTEP, dtype=jnp.int32)
    eff = jnp.minimum(sidx, csum[-1] - 1)
    g = jnp.searchsorted(csum, eff, side="right").astype(jnp.int32)
    t = (first[g] + eff - base[g]).astype(jnp.int32)
    return t, g, starts.astype(jnp.int32), ends.astype(jnp.int32)


def kernel(z, layer_ids, W1, b1, W2, b2):
    ids = layer_ids.astype(jnp.int32)
    sort_idx = jnp.argsort(ids).astype(jnp.int32)
    inv_idx = (jnp.zeros((_N_TOKENS,), jnp.int32)
               .at[sort_idx].set(jnp.arange(_N_TOKENS, dtype=jnp.int32),
                                 mode="drop", unique_indices=True))
    step_t, step_g, starts, ends = _schedule(ids)
    sc_gather = _make_sc_gather(_N_TOKENS, _HIDDEN)
    zs = sc_gather(z, sort_idx)
    ys = _gmm(zs, W1, b1, W2, b2, step_t, step_g, starts, ends)
    return sc_gather(ys, inv_idx)


# speedup vs baseline: 1.4229x; 1.4229x over previous
import functools

import jax
import jax.numpy as jnp
from jax import lax
from jax.experimental import pallas as pl
from jax.experimental.pallas import tpu as pltpu
from jax.experimental.pallas import tpu_sc as plsc

_NUM_LAYERS = 16
_HIDDEN = 768
_INNER = 1536
_N_TOKENS = 32768

_TM = 256
_NTILES = _N_TOKENS // _TM
_NSTEP = _NTILES + _NUM_LAYERS - 1


_NC = 2
_NS = 16
_NW = _NC * _NS


def _make_sc_gather(n_rows, d):
    rpw = n_rows // _NW
    chunk = 64
    nch = rpw // chunk
    mesh = plsc.VectorSubcoreMesh(core_axis_name="c", subcore_axis_name="s")

    @functools.partial(
        pl.kernel,
        mesh=mesh,
        out_type=jax.ShapeDtypeStruct((n_rows, d), jnp.float32),
        scratch_types=[
            pltpu.VMEM((rpw,), jnp.int32),
            pltpu.VMEM((2, chunk, d), jnp.float32),
            pltpu.SemaphoreType.DMA,
            pltpu.SemaphoreType.DMA,
            pltpu.SemaphoreType.DMA,
            pltpu.SemaphoreType.DMA,
        ],
    )
    def gather_k(src_hbm, idx_hbm, out_hbm, idx_v, buf_v, gsem0, gsem1,
                 wsem0, wsem1):
        cid = lax.axis_index("c")
        sid = lax.axis_index("s")
        wid = sid * _NC + cid
        base = wid * rpw
        pltpu.sync_copy(idx_hbm.at[pl.ds(base, rpw)], idx_v)
        gsem = (gsem0, gsem1)
        wsem = (wsem0, wsem1)
        g_cp = [None, None]
        w_cp = [None, None]
        for c in range(nch):
            b = c % 2
            if w_cp[b] is not None:
                w_cp[b].wait()
            g_cp[b] = pltpu.async_copy(
                src_hbm.at[idx_v.at[pl.ds(c * chunk, chunk)]],
                buf_v.at[b], gsem[b])
            if c >= 1:
                pb = (c - 1) % 2
                g_cp[pb].wait()
                w_cp[pb] = pltpu.async_copy(
                    buf_v.at[pb],
                    out_hbm.at[pl.ds(base + (c - 1) * chunk, chunk)],
                    wsem[pb])
        lb = (nch - 1) % 2
        g_cp[lb].wait()
        w_cp[lb] = pltpu.async_copy(
            buf_v.at[lb],
            out_hbm.at[pl.ds(base + (nch - 1) * chunk, chunk)], wsem[lb])
        w_cp[(nch - 2) % 2].wait()
        w_cp[lb].wait()

    return gather_k


_make_sc_gather = functools.lru_cache(maxsize=None)(_make_sc_gather)


def _make_sc_dispatch(n_rows, d):
    rpw = n_rows // _NW
    chunk = 64
    nch = rpw // chunk
    nv = rpw // 16
    mesh = plsc.VectorSubcoreMesh(core_axis_name="c", subcore_axis_name="s")

    @functools.partial(
        pl.kernel,
        mesh=mesh,
        out_type=(
            jax.ShapeDtypeStruct((n_rows, d), jnp.float32),
            jax.ShapeDtypeStruct((n_rows,), jnp.int32),
        ),
        scratch_types=[
            pltpu.VMEM((rpw,), jnp.int32),
            pltpu.VMEM((16, 16), jnp.int32),
            pltpu.VMEM((rpw,), jnp.int32),
            pltpu.VMEM((2, chunk, d), jnp.float32),
            pltpu.SemaphoreType.DMA,
            pltpu.SemaphoreType.DMA,
            pltpu.SemaphoreType.DMA,
            pltpu.SemaphoreType.DMA,
        ],
    )
    def dispatch_k(src_hbm, ids_hbm, base_hbm, zs_hbm, p_hbm,
                   ids_v, base_v, p_v, buf_v, gsem0, gsem1, ssem0, ssem1):
        cid = lax.axis_index("c")
        sid = lax.axis_index("s")
        wid = sid * _NC + cid
        base = wid * rpw
        pltpu.sync_copy(ids_hbm.at[pl.ds(base, rpw)], ids_v)
        pltpu.sync_copy(base_hbm.at[wid], base_v)

        cnt0 = tuple(base_v[l] for l in range(_NUM_LAYERS))

        ones = jnp.full((16,), 1, jnp.int32)
        zeros = jnp.full((16,), 0, jnp.int32)

        def vbody(v, cnts):
            idsv = ids_v[pl.ds(v * 16, 16)]
            slot = zeros
            new = []
            for l in range(_NUM_LAYERS):
                m = idsv == jnp.full((16,), l, jnp.int32)
                slot = jnp.where(m, cnts[l], slot)
                new.append(cnts[l] + jnp.where(m, ones, zeros))
            p_v[pl.ds(v * 16, 16)] = slot
            return tuple(new)

        lax.fori_loop(0, nv, vbody, cnt0)
        pltpu.sync_copy(p_v, p_hbm.at[pl.ds(base, rpw)])

        gsem = (gsem0, gsem1)
        ssem = (ssem0, ssem1)
        g_cp = [None, None]
        s_cp = [[], []]

        def scatter_chunk(c):
            b = c % 2
            for q in range(chunk // 16):
                pv = p_v[pl.ds(c * chunk + q * 16, 16)]
                s_cp[b].append(pltpu.async_copy(
                    buf_v.at[b].at[pl.ds(q * 16, 16)],
                    zs_hbm.at[pv], ssem[b]))

        for c in range(nch):
            b = c % 2
            for cp in s_cp[b]:
                cp.wait()
            s_cp[b] = []
            g_cp[b] = pltpu.async_copy(
                src_hbm.at[pl.ds(base + c * chunk, chunk)], buf_v.at[b],
                gsem[b])
            if c >= 1:
                g_cp[(c - 1) % 2].wait()
                scatter_chunk(c - 1)
        g_cp[(nch - 1) % 2].wait()
        scatter_chunk(nch - 1)
        for b in (0, 1):
            for cp in s_cp[b]:
                cp.wait()

    return dispatch_k


_make_sc_dispatch = functools.lru_cache(maxsize=None)(_make_sc_dispatch)


def _gelu(x):
    return 0.5 * x * (1.0 + lax.erf(x * (2.0 ** -0.5)))


def _gmm_body(st_ref, sg_ref, starts_ref, ends_ref,
              zs_ref, w1_ref, b1_ref, w2_ref, b2_ref, out_ref):
    i = pl.program_id(0)
    t = st_ref[i]
    g = sg_ref[i]
    rows = t * _TM + lax.broadcasted_iota(jnp.int32, (_TM, 1), 0)
    mask = (rows >= starts_ref[g]) & (rows < ends_ref[g])
    x = zs_ref[...]
    h = lax.dot_general(x, w1_ref[0], (((1,), (1,)), ((), ())),
                        preferred_element_type=jnp.float32)
    h = _gelu(h + b1_ref[0])
    y = lax.dot_general(h, w2_ref[0], (((1,), (1,)), ((), ())),
                        preferred_element_type=jnp.float32)
    y = y + b2_ref[0] + x
    out_ref[...] = jnp.where(mask, y, out_ref[...])


def _gmm(zs, W1, b1, W2, b2, step_t, step_g, starts, ends):
    grid_spec = pltpu.PrefetchScalarGridSpec(
        num_scalar_prefetch=4,
        grid=(_NSTEP,),
        in_specs=[
            pl.BlockSpec((_TM, _HIDDEN), lambda i, st, sg, s0, e0: (st[i], 0)),
            pl.BlockSpec((1, _INNER, _HIDDEN),
                         lambda i, st, sg, s0, e0: (sg[i], 0, 0)),
            pl.BlockSpec((1, 1, _INNER), lambda i, st, sg, s0, e0: (sg[i], 0, 0)),
            pl.BlockSpec((1, _HIDDEN, _INNER),
                         lambda i, st, sg, s0, e0: (sg[i], 0, 0)),
            pl.BlockSpec((1, 1, _HIDDEN), lambda i, st, sg, s0, e0: (sg[i], 0, 0)),
        ],
        out_specs=pl.BlockSpec((_TM, _HIDDEN),
                               lambda i, st, sg, s0, e0: (st[i], 0)),
    )
    return pl.pallas_call(
        _gmm_body,
        grid_spec=grid_spec,
        out_shape=jax.ShapeDtypeStruct((_N_TOKENS, _HIDDEN), jnp.float32),
        compiler_params=pltpu.CompilerParams(
            dimension_semantics=("arbitrary",)),
    )(step_t, step_g, starts, ends, zs, W1,
      b1.reshape(_NUM_LAYERS, 1, _INNER), W2,
      b2.reshape(_NUM_LAYERS, 1, _HIDDEN))


def _schedule(sizes):
    ends = jnp.cumsum(sizes)
    starts = ends - sizes
    first = starts // _TM
    count = jnp.where(sizes > 0, (ends - 1) // _TM - first + 1, 0)
    csum = jnp.cumsum(count)
    base = csum - count
    sidx = jnp.arange(_NSTEP, dtype=jnp.int32)
    eff = jnp.minimum(sidx, csum[-1] - 1)
    g = jnp.searchsorted(csum, eff, side="right").astype(jnp.int32)
    t = (first[g] + eff - base[g]).astype(jnp.int32)
    return t, g, starts.astype(jnp.int32), ends.astype(jnp.int32)


def kernel(z, layer_ids, W1, b1, W2, b2):
    ids = layer_ids.astype(jnp.int32)
    rpw = _N_TOKENS // _NW
    i = jnp.arange(_N_TOKENS, dtype=jnp.int32)
    w = i // rpw
    j = i % 16
    hist3 = (jnp.zeros((_NUM_LAYERS, _NW, 16), jnp.int32)
             .at[ids, w, j].add(1, mode="drop"))
    sizes = jnp.sum(hist3, axis=(1, 2))
    step_t, step_g, starts, ends = _schedule(sizes)
    flat = hist3.reshape(_NUM_LAYERS, _NW * 16)
    basef = starts[:, None] + jnp.cumsum(flat, axis=1) - flat
    base3 = (basef.reshape(_NUM_LAYERS, _NW, 16)
             .transpose(1, 0, 2).astype(jnp.int32))
    zs, p = _make_sc_dispatch(_N_TOKENS, _HIDDEN)(z, ids, base3)
    ys = _gmm(zs, W1, b1, W2, b2, step_t, step_g, starts, ends)
    return _make_sc_gather(_N_TOKENS, _HIDDEN)(ys, p)


# speedup vs baseline: 1.4269x; 1.0028x over previous
import functools

import jax
import jax.numpy as jnp
from jax import lax
from jax.experimental import pallas as pl
from jax.experimental.pallas import tpu as pltpu
from jax.experimental.pallas import tpu_sc as plsc

_NUM_LAYERS = 16
_HIDDEN = 768
_INNER = 1536
_N_TOKENS = 32768

_TM = 256
_NTILES = _N_TOKENS // _TM
_NSTEP = _NTILES + _NUM_LAYERS - 1


_NC = 2
_NS = 16
_NW = _NC * _NS


def _make_sc_gather(n_rows, d):
    rpw = n_rows // _NW
    chunk = 64
    nch = rpw // chunk
    mesh = plsc.VectorSubcoreMesh(core_axis_name="c", subcore_axis_name="s")

    @functools.partial(
        pl.kernel,
        mesh=mesh,
        out_type=jax.ShapeDtypeStruct((n_rows, d), jnp.float32),
        scratch_types=[
            pltpu.VMEM((rpw,), jnp.int32),
            pltpu.VMEM((2, chunk, d), jnp.float32),
            pltpu.SemaphoreType.DMA,
            pltpu.SemaphoreType.DMA,
            pltpu.SemaphoreType.DMA,
            pltpu.SemaphoreType.DMA,
        ],
    )
    def gather_k(src_hbm, idx_hbm, out_hbm, idx_v, buf_v, gsem0, gsem1,
                 wsem0, wsem1):
        cid = lax.axis_index("c")
        sid = lax.axis_index("s")
        wid = sid * _NC + cid
        base = wid * rpw
        pltpu.sync_copy(idx_hbm.at[pl.ds(base, rpw)], idx_v)
        gsem = (gsem0, gsem1)
        wsem = (wsem0, wsem1)
        g_cp = [None, None]
        w_cp = [None, None]
        for c in range(nch):
            b = c % 2
            if w_cp[b] is not None:
                w_cp[b].wait()
            g_cp[b] = pltpu.async_copy(
                src_hbm.at[idx_v.at[pl.ds(c * chunk, chunk)]],
                buf_v.at[b], gsem[b])
            if c >= 1:
                pb = (c - 1) % 2
                g_cp[pb].wait()
                w_cp[pb] = pltpu.async_copy(
                    buf_v.at[pb],
                    out_hbm.at[pl.ds(base + (c - 1) * chunk, chunk)],
                    wsem[pb])
        lb = (nch - 1) % 2
        g_cp[lb].wait()
        w_cp[lb] = pltpu.async_copy(
            buf_v.at[lb],
            out_hbm.at[pl.ds(base + (nch - 1) * chunk, chunk)], wsem[lb])
        w_cp[(nch - 2) % 2].wait()
        w_cp[lb].wait()

    return gather_k


_make_sc_gather = functools.lru_cache(maxsize=None)(_make_sc_gather)


def _make_sc_dispatch(n_rows, d):
    rpw = n_rows // _NW
    chunk = 64
    nch = rpw // chunk
    nv = rpw // 16
    mesh = plsc.VectorSubcoreMesh(core_axis_name="c", subcore_axis_name="s")

    @functools.partial(
        pl.kernel,
        mesh=mesh,
        out_type=(
            jax.ShapeDtypeStruct((n_rows, d), jnp.float32),
            jax.ShapeDtypeStruct((n_rows,), jnp.int32),
        ),
        scratch_types=[
            pltpu.VMEM((rpw,), jnp.int32),
            pltpu.VMEM((16, 16), jnp.int32),
            pltpu.VMEM((rpw,), jnp.int32),
            pltpu.VMEM((2, chunk, d), jnp.float32),
            pltpu.SemaphoreType.DMA,
            pltpu.SemaphoreType.DMA,
            pltpu.SemaphoreType.DMA,
            pltpu.SemaphoreType.DMA,
        ],
    )
    def dispatch_k(src_hbm, ids_hbm, base_hbm, zs_hbm, p_hbm,
                   ids_v, base_v, p_v, buf_v, gsem0, gsem1, ssem0, ssem1):
        cid = lax.axis_index("c")
        sid = lax.axis_index("s")
        wid = sid * _NC + cid
        base = wid * rpw
        pltpu.sync_copy(ids_hbm.at[pl.ds(base, rpw)], ids_v)
        pltpu.sync_copy(base_hbm.at[wid], base_v)

        cnt0 = tuple(base_v[l] for l in range(_NUM_LAYERS))

        ones = jnp.full((16,), 1, jnp.int32)
        zeros = jnp.full((16,), 0, jnp.int32)

        def vbody(v, cnts):
            idsv = ids_v[pl.ds(v * 16, 16)]
            slot = zeros
            new = []
            for l in range(_NUM_LAYERS):
                m = idsv == jnp.full((16,), l, jnp.int32)
                slot = jnp.where(m, cnts[l], slot)
                new.append(cnts[l] + jnp.where(m, ones, zeros))
            p_v[pl.ds(v * 16, 16)] = slot
            return tuple(new)

        lax.fori_loop(0, nv, vbody, cnt0)
        pltpu.sync_copy(p_v, p_hbm.at[pl.ds(base, rpw)])

        gsem = (gsem0, gsem1)
        ssem = (ssem0, ssem1)
        g_cp = [None, None]
        s_cp = [[], []]

        def scatter_chunk(c):
            b = c % 2
            for q in range(chunk // 16):
                pv = p_v[pl.ds(c * chunk + q * 16, 16)]
                s_cp[b].append(pltpu.async_copy(
                    buf_v.at[b].at[pl.ds(q * 16, 16)],
                    zs_hbm.at[pv], ssem[b]))

        for c in range(nch):
            b = c % 2
            for cp in s_cp[b]:
                cp.wait()
            s_cp[b] = []
            g_cp[b] = pltpu.async_copy(
                src_hbm.at[pl.ds(base + c * chunk, chunk)], buf_v.at[b],
                gsem[b])
            if c >= 1:
                g_cp[(c - 1) % 2].wait()
                scatter_chunk(c - 1)
        g_cp[(nch - 1) % 2].wait()
        scatter_chunk(nch - 1)
        for b in (0, 1):
            for cp in s_cp[b]:
                cp.wait()

    return dispatch_k


_make_sc_dispatch = functools.lru_cache(maxsize=None)(_make_sc_dispatch)


def _gelu(x):
    return 0.5 * x * (1.0 + lax.erf(x * (2.0 ** -0.5)))


def _gmm_body(st_ref, sg_ref, starts_ref, ends_ref,
              zs_ref, w1_ref, b1_ref, w2_ref, b2_ref, out_ref):
    i = pl.program_id(0)
    t = st_ref[i]
    g = sg_ref[i]
    rows = t * _TM + lax.broadcasted_iota(jnp.int32, (_TM, 1), 0)
    mask = (rows >= starts_ref[g]) & (rows < ends_ref[g])
    x = zs_ref[...]
    h = lax.dot_general(x, w1_ref[0], (((1,), (1,)), ((), ())),
                        preferred_element_type=jnp.float32)
    h = _gelu(h + b1_ref[0])
    y = lax.dot_general(h, w2_ref[0], (((1,), (1,)), ((), ())),
                        preferred_element_type=jnp.float32)
    y = y + b2_ref[0] + x
    out_ref[...] = jnp.where(mask, y, out_ref[...])


def _gmm(zs, W1, b1, W2, b2, step_t, step_g, starts, ends):
    grid_spec = pltpu.PrefetchScalarGridSpec(
        num_scalar_prefetch=4,
        grid=(_NSTEP,),
        in_specs=[
            pl.BlockSpec((_TM, _HIDDEN), lambda i, st, sg, s0, e0: (st[i], 0)),
            pl.BlockSpec((1, _INNER, _HIDDEN),
                         lambda i, st, sg, s0, e0: (sg[i], 0, 0)),
            pl.BlockSpec((1, 1, _INNER), lambda i, st, sg, s0, e0: (sg[i], 0, 0)),
            pl.BlockSpec((1, _HIDDEN, _INNER),
                         lambda i, st, sg, s0, e0: (sg[i], 0, 0)),
            pl.BlockSpec((1, 1, _HIDDEN), lambda i, st, sg, s0, e0: (sg[i], 0, 0)),
        ],
        out_specs=pl.BlockSpec((_TM, _HIDDEN),
                               lambda i, st, sg, s0, e0: (st[i], 0)),
    )
    return pl.pallas_call(
        _gmm_body,
        grid_spec=grid_spec,
        out_shape=jax.ShapeDtypeStruct((_N_TOKENS, _HIDDEN), jnp.float32),
        compiler_params=pltpu.CompilerParams(
            dimension_semantics=("arbitrary",)),
    )(step_t, step_g, starts, ends, zs, W1,
      b1.reshape(_NUM_LAYERS, 1, _INNER), W2,
      b2.reshape(_NUM_LAYERS, 1, _HIDDEN))


def _schedule(sizes):
    ends = jnp.cumsum(sizes)
    starts = ends - sizes
    first = starts // _TM
    count = jnp.where(sizes > 0, (ends - 1) // _TM - first + 1, 0)
    csum = jnp.cumsum(count)
    base = csum - count
    sidx = jnp.arange(_NSTEP, dtype=jnp.int32)
    eff = jnp.minimum(sidx, csum[-1] - 1)
    g = jnp.searchsorted(csum, eff, side="right").astype(jnp.int32)
    t = (first[g] + eff - base[g]).astype(jnp.int32)
    return t, g, starts.astype(jnp.int32), ends.astype(jnp.int32)


def kernel(z, layer_ids, W1, b1, W2, b2):
    ids = layer_ids.astype(jnp.int32)
    rpw = _N_TOKENS // _NW
    i = jnp.arange(_N_TOKENS, dtype=jnp.int32)
    w = i // rpw
    j = i % 16
    hist3 = (jnp.zeros((_NUM_LAYERS, _NW, 16), jnp.int32)
             .at[ids, w, j].add(1, mode="drop"))
    sizes = jnp.sum(hist3, axis=(1, 2))
    step_t, step_g, starts, ends = _schedule(sizes)
    flat = hist3.reshape(_NUM_LAYERS, _NW * 16)
    triu = jnp.triu(jnp.ones((_NW * 16, _NW * 16), jnp.float32))
    csum = jnp.dot(flat.astype(jnp.float32), triu,
                   preferred_element_type=jnp.float32).astype(jnp.int32)
    basef = starts[:, None] + csum - flat
    base3 = (basef.reshape(_NUM_LAYERS, _NW, 16)
             .transpose(1, 0, 2).astype(jnp.int32))
    zs, p = _make_sc_dispatch(_N_TOKENS, _HIDDEN)(z, ids, base3)
    ys = _gmm(zs, W1, b1, W2, b2, step_t, step_g, starts, ends)
    return _make_sc_gather(_N_TOKENS, _HIDDEN)(ys, p)


# speedup vs baseline: 1.6420x; 1.1508x over previous
import functools

import jax
import jax.numpy as jnp
from jax import lax
from jax.experimental import pallas as pl
from jax.experimental.pallas import tpu as pltpu
from jax.experimental.pallas import tpu_sc as plsc

_NUM_LAYERS = 16
_HIDDEN = 768
_INNER = 1536
_N_TOKENS = 32768

_TM = 512
_NTILES = _N_TOKENS // _TM
_NSTEP = _NTILES + _NUM_LAYERS - 1


_NC = 2
_NS = 16
_NW = _NC * _NS


def _make_sc_gather(n_rows, d):
    rpw = n_rows // _NW
    chunk = 64
    nch = rpw // chunk
    mesh = plsc.VectorSubcoreMesh(core_axis_name="c", subcore_axis_name="s")

    @functools.partial(
        pl.kernel,
        mesh=mesh,
        out_type=jax.ShapeDtypeStruct((n_rows, d), jnp.float32),
        scratch_types=[
            pltpu.VMEM((rpw,), jnp.int32),
            pltpu.VMEM((2, chunk, d), jnp.float32),
            pltpu.SemaphoreType.DMA,
            pltpu.SemaphoreType.DMA,
            pltpu.SemaphoreType.DMA,
            pltpu.SemaphoreType.DMA,
        ],
    )
    def gather_k(src_hbm, idx_hbm, out_hbm, idx_v, buf_v, gsem0, gsem1,
                 wsem0, wsem1):
        cid = lax.axis_index("c")
        sid = lax.axis_index("s")
        wid = sid * _NC + cid
        base = wid * rpw
        pltpu.sync_copy(idx_hbm.at[pl.ds(base, rpw)], idx_v)
        gsem = (gsem0, gsem1)
        wsem = (wsem0, wsem1)
        g_cp = [None, None]
        w_cp = [None, None]
        for c in range(nch):
            b = c % 2
            if w_cp[b] is not None:
                w_cp[b].wait()
            g_cp[b] = pltpu.async_copy(
                src_hbm.at[idx_v.at[pl.ds(c * chunk, chunk)]],
                buf_v.at[b], gsem[b])
            if c >= 1:
                pb = (c - 1) % 2
                g_cp[pb].wait()
                w_cp[pb] = pltpu.async_copy(
                    buf_v.at[pb],
                    out_hbm.at[pl.ds(base + (c - 1) * chunk, chunk)],
                    wsem[pb])
        lb = (nch - 1) % 2
        g_cp[lb].wait()
        w_cp[lb] = pltpu.async_copy(
            buf_v.at[lb],
            out_hbm.at[pl.ds(base + (nch - 1) * chunk, chunk)], wsem[lb])
        w_cp[(nch - 2) % 2].wait()
        w_cp[lb].wait()

    return gather_k


_make_sc_gather = functools.lru_cache(maxsize=None)(_make_sc_gather)


def _make_sc_dispatch(n_rows, d):
    rpw = n_rows // _NW
    chunk = 64
    nch = rpw // chunk
    nv = rpw // 16
    mesh = plsc.VectorSubcoreMesh(core_axis_name="c", subcore_axis_name="s")

    @functools.partial(
        pl.kernel,
        mesh=mesh,
        out_type=(
            jax.ShapeDtypeStruct((n_rows, d), jnp.float32),
            jax.ShapeDtypeStruct((n_rows,), jnp.int32),
        ),
        scratch_types=[
            pltpu.VMEM((rpw,), jnp.int32),
            pltpu.VMEM((16, 16), jnp.int32),
            pltpu.VMEM((rpw,), jnp.int32),
            pltpu.VMEM((2, chunk, d), jnp.float32),
            pltpu.SemaphoreType.DMA,
            pltpu.SemaphoreType.DMA,
            pltpu.SemaphoreType.DMA,
            pltpu.SemaphoreType.DMA,
        ],
    )
    def dispatch_k(src_hbm, ids_hbm, base_hbm, zs_hbm, p_hbm,
                   ids_v, base_v, p_v, buf_v, gsem0, gsem1, ssem0, ssem1):
        cid = lax.axis_index("c")
        sid = lax.axis_index("s")
        wid = sid * _NC + cid
        base = wid * rpw
        pltpu.sync_copy(ids_hbm.at[pl.ds(base, rpw)], ids_v)
        pltpu.sync_copy(base_hbm.at[wid], base_v)

        cnt0 = tuple(base_v[l] for l in range(_NUM_LAYERS))

        ones = jnp.full((16,), 1, jnp.int32)
        zeros = jnp.full((16,), 0, jnp.int32)

        def vbody(v, cnts):
            idsv = ids_v[pl.ds(v * 16, 16)]
            slot = zeros
            new = []
            for l in range(_NUM_LAYERS):
                m = idsv == jnp.full((16,), l, jnp.int32)
                slot = jnp.where(m, cnts[l], slot)
                new.append(cnts[l] + jnp.where(m, ones, zeros))
            p_v[pl.ds(v * 16, 16)] = slot
            return tuple(new)

        lax.fori_loop(0, nv, vbody, cnt0)
        pltpu.sync_copy(p_v, p_hbm.at[pl.ds(base, rpw)])

        gsem = (gsem0, gsem1)
        ssem = (ssem0, ssem1)
        g_cp = [None, None]
        s_cp = [[], []]

        def scatter_chunk(c):
            b = c % 2
            for q in range(chunk // 16):
                pv = p_v[pl.ds(c * chunk + q * 16, 16)]
                s_cp[b].append(pltpu.async_copy(
                    buf_v.at[b].at[pl.ds(q * 16, 16)],
                    zs_hbm.at[pv], ssem[b]))

        for c in range(nch):
            b = c % 2
            for cp in s_cp[b]:
                cp.wait()
            s_cp[b] = []
            g_cp[b] = pltpu.async_copy(
                src_hbm.at[pl.ds(base + c * chunk, chunk)], buf_v.at[b],
                gsem[b])
            if c >= 1:
                g_cp[(c - 1) % 2].wait()
                scatter_chunk(c - 1)
        g_cp[(nch - 1) % 2].wait()
        scatter_chunk(nch - 1)
        for b in (0, 1):
            for cp in s_cp[b]:
                cp.wait()

    return dispatch_k


_make_sc_dispatch = functools.lru_cache(maxsize=None)(_make_sc_dispatch)


def _gelu(x):
    return 0.5 * x * (1.0 + lax.erf(x * (2.0 ** -0.5)))


def _gmm_body(st_ref, sg_ref, starts_ref, ends_ref,
              zs_ref, w1_ref, b1_ref, w2_ref, b2_ref, out_ref):
    i = pl.program_id(0)
    t = st_ref[i]
    g = sg_ref[i]
    rows = t * _TM + lax.broadcasted_iota(jnp.int32, (_TM, 1), 0)
    mask = (rows >= starts_ref[g]) & (rows < ends_ref[g])
    x = zs_ref[...]
    h = lax.dot_general(x, w1_ref[0], (((1,), (1,)), ((), ())),
                        preferred_element_type=jnp.float32)
    h = _gelu(h + b1_ref[0])
    y = lax.dot_general(h, w2_ref[0], (((1,), (1,)), ((), ())),
                        preferred_element_type=jnp.float32)
    y = y + b2_ref[0] + x
    out_ref[...] = jnp.where(mask, y, out_ref[...])


def _gmm(zs, W1, b1, W2, b2, step_t, step_g, starts, ends):
    grid_spec = pltpu.PrefetchScalarGridSpec(
        num_scalar_prefetch=4,
        grid=(_NSTEP,),
        in_specs=[
            pl.BlockSpec((_TM, _HIDDEN), lambda i, st, sg, s0, e0: (st[i], 0)),
            pl.BlockSpec((1, _INNER, _HIDDEN),
                         lambda i, st, sg, s0, e0: (sg[i], 0, 0)),
            pl.BlockSpec((1, 1, _INNER), lambda i, st, sg, s0, e0: (sg[i], 0, 0)),
            pl.BlockSpec((1, _HIDDEN, _INNER),
                         lambda i, st, sg, s0, e0: (sg[i], 0, 0)),
            pl.BlockSpec((1, 1, _HIDDEN), lambda i, st, sg, s0, e0: (sg[i], 0, 0)),
        ],
        out_specs=pl.BlockSpec((_TM, _HIDDEN),
                               lambda i, st, sg, s0, e0: (st[i], 0)),
    )
    return pl.pallas_call(
        _gmm_body,
        grid_spec=grid_spec,
        out_shape=jax.ShapeDtypeStruct((_N_TOKENS, _HIDDEN), jnp.float32),
        compiler_params=pltpu.CompilerParams(
            dimension_semantics=("arbitrary",)),
    )(step_t, step_g, starts, ends, zs, W1,
      b1.reshape(_NUM_LAYERS, 1, _INNER), W2,
      b2.reshape(_NUM_LAYERS, 1, _HIDDEN))


def _schedule(sizes):
    ends = jnp.cumsum(sizes)
    starts = ends - sizes
    first = starts // _TM
    count = jnp.where(sizes > 0, (ends - 1) // _TM - first + 1, 0)
    csum = jnp.cumsum(count)
    base = csum - count
    sidx = jnp.arange(_NSTEP, dtype=jnp.int32)
    eff = jnp.minimum(sidx, csum[-1] - 1)
    g = jnp.searchsorted(csum, eff, side="right").astype(jnp.int32)
    t = (first[g] + eff - base[g]).astype(jnp.int32)
    return t, g, starts.astype(jnp.int32), ends.astype(jnp.int32)


def kernel(z, layer_ids, W1, b1, W2, b2):
    ids = layer_ids.astype(jnp.int32)
    rpw = _N_TOKENS // _NW
    i = jnp.arange(_N_TOKENS, dtype=jnp.int32)
    w = i // rpw
    j = i % 16
    hist3 = (jnp.zeros((_NUM_LAYERS, _NW, 16), jnp.int32)
             .at[ids, w, j].add(1, mode="drop"))
    sizes = jnp.sum(hist3, axis=(1, 2))
    step_t, step_g, starts, ends = _schedule(sizes)
    flat = hist3.reshape(_NUM_LAYERS, _NW * 16)
    triu = jnp.triu(jnp.ones((_NW * 16, _NW * 16), jnp.float32))
    csum = jnp.dot(flat.astype(jnp.float32), triu,
                   preferred_element_type=jnp.float32).astype(jnp.int32)
    basef = starts[:, None] + csum - flat
    base3 = (basef.reshape(_NUM_LAYERS, _NW, 16)
             .transpose(1, 0, 2).astype(jnp.int32))
    zs, p = _make_sc_dispatch(_N_TOKENS, _HIDDEN)(z, ids, base3)
    ys = _gmm(zs, W1, b1, W2, b2, step_t, step_g, starts, ends)
    return _make_sc_gather(_N_TOKENS, _HIDDEN)(ys, p)


# speedup vs baseline: 1.6473x; 1.0032x over previous
import functools

import jax
import jax.numpy as jnp
from jax import lax
from jax.experimental import pallas as pl
from jax.experimental.pallas import tpu as pltpu
from jax.experimental.pallas import tpu_sc as plsc

_NUM_LAYERS = 16
_HIDDEN = 768
_INNER = 1536
_N_TOKENS = 32768

_TM = 1024
_NTILES = _N_TOKENS // _TM
_NSTEP = _NTILES + _NUM_LAYERS - 1


_NC = 2
_NS = 16
_NW = _NC * _NS


def _make_sc_gather(n_rows, d):
    rpw = n_rows // _NW
    chunk = 64
    nch = rpw // chunk
    mesh = plsc.VectorSubcoreMesh(core_axis_name="c", subcore_axis_name="s")

    @functools.partial(
        pl.kernel,
        mesh=mesh,
        out_type=jax.ShapeDtypeStruct((n_rows, d), jnp.float32),
        scratch_types=[
            pltpu.VMEM((rpw,), jnp.int32),
            pltpu.VMEM((2, chunk, d), jnp.float32),
            pltpu.SemaphoreType.DMA,
            pltpu.SemaphoreType.DMA,
            pltpu.SemaphoreType.DMA,
            pltpu.SemaphoreType.DMA,
        ],
    )
    def gather_k(src_hbm, idx_hbm, out_hbm, idx_v, buf_v, gsem0, gsem1,
                 wsem0, wsem1):
        cid = lax.axis_index("c")
        sid = lax.axis_index("s")
        wid = sid * _NC + cid
        base = wid * rpw
        pltpu.sync_copy(idx_hbm.at[pl.ds(base, rpw)], idx_v)
        gsem = (gsem0, gsem1)
        wsem = (wsem0, wsem1)
        g_cp = [None, None]
        w_cp = [None, None]
        for c in range(nch):
            b = c % 2
            if w_cp[b] is not None:
                w_cp[b].wait()
            g_cp[b] = pltpu.async_copy(
                src_hbm.at[idx_v.at[pl.ds(c * chunk, chunk)]],
                buf_v.at[b], gsem[b])
            if c >= 1:
                pb = (c - 1) % 2
                g_cp[pb].wait()
                w_cp[pb] = pltpu.async_copy(
                    buf_v.at[pb],
                    out_hbm.at[pl.ds(base + (c - 1) * chunk, chunk)],
                    wsem[pb])
        lb = (nch - 1) % 2
        g_cp[lb].wait()
        w_cp[lb] = pltpu.async_copy(
            buf_v.at[lb],
            out_hbm.at[pl.ds(base + (nch - 1) * chunk, chunk)], wsem[lb])
        w_cp[(nch - 2) % 2].wait()
        w_cp[lb].wait()

    return gather_k


_make_sc_gather = functools.lru_cache(maxsize=None)(_make_sc_gather)


def _make_sc_dispatch(n_rows, d):
    rpw = n_rows // _NW
    chunk = 64
    nch = rpw // chunk
    nv = rpw // 16
    mesh = plsc.VectorSubcoreMesh(core_axis_name="c", subcore_axis_name="s")

    @functools.partial(
        pl.kernel,
        mesh=mesh,
        out_type=(
            jax.ShapeDtypeStruct((n_rows, d), jnp.float32),
            jax.ShapeDtypeStruct((n_rows,), jnp.int32),
        ),
        scratch_types=[
            pltpu.VMEM((rpw,), jnp.int32),
            pltpu.VMEM((16, 16), jnp.int32),
            pltpu.VMEM((rpw,), jnp.int32),
            pltpu.VMEM((2, chunk, d), jnp.float32),
            pltpu.SemaphoreType.DMA,
            pltpu.SemaphoreType.DMA,
            pltpu.SemaphoreType.DMA,
            pltpu.SemaphoreType.DMA,
        ],
    )
    def dispatch_k(src_hbm, ids_hbm, base_hbm, zs_hbm, p_hbm,
                   ids_v, base_v, p_v, buf_v, gsem0, gsem1, ssem0, ssem1):
        cid = lax.axis_index("c")
        sid = lax.axis_index("s")
        wid = sid * _NC + cid
        base = wid * rpw
        pltpu.sync_copy(ids_hbm.at[pl.ds(base, rpw)], ids_v)
        pltpu.sync_copy(base_hbm.at[wid], base_v)

        cnt0 = tuple(base_v[l] for l in range(_NUM_LAYERS))

        ones = jnp.full((16,), 1, jnp.int32)
        zeros = jnp.full((16,), 0, jnp.int32)

        def vbody(v, cnts):
            idsv = ids_v[pl.ds(v * 16, 16)]
            slot = zeros
            new = []
            for l in range(_NUM_LAYERS):
                m = idsv == jnp.full((16,), l, jnp.int32)
                slot = jnp.where(m, cnts[l], slot)
                new.append(cnts[l] + jnp.where(m, ones, zeros))
            p_v[pl.ds(v * 16, 16)] = slot
            return tuple(new)

        lax.fori_loop(0, nv, vbody, cnt0)
        pltpu.sync_copy(p_v, p_hbm.at[pl.ds(base, rpw)])

        gsem = (gsem0, gsem1)
        ssem = (ssem0, ssem1)
        g_cp = [None, None]
        s_cp = [[], []]

        def scatter_chunk(c):
            b = c % 2
            for q in range(chunk // 16):
                pv = p_v[pl.ds(c * chunk + q * 16, 16)]
                s_cp[b].append(pltpu.async_copy(
                    buf_v.at[b].at[pl.ds(q * 16, 16)],
                    zs_hbm.at[pv], ssem[b]))

        for c in range(nch):
            b = c % 2
            for cp in s_cp[b]:
                cp.wait()
            s_cp[b] = []
            g_cp[b] = pltpu.async_copy(
                src_hbm.at[pl.ds(base + c * chunk, chunk)], buf_v.at[b],
                gsem[b])
            if c >= 1:
                g_cp[(c - 1) % 2].wait()
                scatter_chunk(c - 1)
        g_cp[(nch - 1) % 2].wait()
        scatter_chunk(nch - 1)
        for b in (0, 1):
            for cp in s_cp[b]:
                cp.wait()

    return dispatch_k


_make_sc_dispatch = functools.lru_cache(maxsize=None)(_make_sc_dispatch)


def _gelu(x):
    return 0.5 * x * (1.0 + lax.erf(x * (2.0 ** -0.5)))


def _gmm_body(st_ref, sg_ref, starts_ref, ends_ref,
              zs_ref, w1_ref, b1_ref, w2_ref, b2_ref, out_ref):
    i = pl.program_id(0)
    t = st_ref[i]
    g = sg_ref[i]
    rows = t * _TM + lax.broadcasted_iota(jnp.int32, (_TM, 1), 0)
    mask = (rows >= starts_ref[g]) & (rows < ends_ref[g])
    x = zs_ref[...]
    h = lax.dot_general(x, w1_ref[0], (((1,), (1,)), ((), ())),
                        preferred_element_type=jnp.float32)
    h = _gelu(h + b1_ref[0])
    y = lax.dot_general(h, w2_ref[0], (((1,), (1,)), ((), ())),
                        preferred_element_type=jnp.float32)
    y = y + b2_ref[0] + x
    out_ref[...] = jnp.where(mask, y, out_ref[...])


def _gmm(zs, W1, b1, W2, b2, step_t, step_g, starts, ends):
    grid_spec = pltpu.PrefetchScalarGridSpec(
        num_scalar_prefetch=4,
        grid=(_NSTEP,),
        in_specs=[
            pl.BlockSpec((_TM, _HIDDEN), lambda i, st, sg, s0, e0: (st[i], 0)),
            pl.BlockSpec((1, _INNER, _HIDDEN),
                         lambda i, st, sg, s0, e0: (sg[i], 0, 0)),
            pl.BlockSpec((1, 1, _INNER), lambda i, st, sg, s0, e0: (sg[i], 0, 0)),
            pl.BlockSpec((1, _HIDDEN, _INNER),
                         lambda i, st, sg, s0, e0: (sg[i], 0, 0)),
            pl.BlockSpec((1, 1, _HIDDEN), lambda i, st, sg, s0, e0: (sg[i], 0, 0)),
        ],
        out_specs=pl.BlockSpec((_TM, _HIDDEN),
                               lambda i, st, sg, s0, e0: (st[i], 0)),
    )
    return pl.pallas_call(
        _gmm_body,
        grid_spec=grid_spec,
        out_shape=jax.ShapeDtypeStruct((_N_TOKENS, _HIDDEN), jnp.float32),
        compiler_params=pltpu.CompilerParams(
            dimension_semantics=("arbitrary",)),
    )(step_t, step_g, starts, ends, zs, W1,
      b1.reshape(_NUM_LAYERS, 1, _INNER), W2,
      b2.reshape(_NUM_LAYERS, 1, _HIDDEN))


def _schedule(sizes):
    ends = jnp.cumsum(sizes)
    starts = ends - sizes
    first = starts // _TM
    count = jnp.where(sizes > 0, (ends - 1) // _TM - first + 1, 0)
    csum = jnp.cumsum(count)
    base = csum - count
    sidx = jnp.arange(_NSTEP, dtype=jnp.int32)
    eff = jnp.minimum(sidx, csum[-1] - 1)
    g = jnp.searchsorted(csum, eff, side="right").astype(jnp.int32)
    t = (first[g] + eff - base[g]).astype(jnp.int32)
    return t, g, starts.astype(jnp.int32), ends.astype(jnp.int32)


def kernel(z, layer_ids, W1, b1, W2, b2):
    ids = layer_ids.astype(jnp.int32)
    rpw = _N_TOKENS // _NW
    i = jnp.arange(_N_TOKENS, dtype=jnp.int32)
    w = i // rpw
    j = i % 16
    hist3 = (jnp.zeros((_NUM_LAYERS, _NW, 16), jnp.int32)
             .at[ids, w, j].add(1, mode="drop"))
    sizes = jnp.sum(hist3, axis=(1, 2))
    step_t, step_g, starts, ends = _schedule(sizes)
    flat = hist3.reshape(_NUM_LAYERS, _NW * 16)
    triu = jnp.triu(jnp.ones((_NW * 16, _NW * 16), jnp.float32))
    csum = jnp.dot(flat.astype(jnp.float32), triu,
                   preferred_element_type=jnp.float32).astype(jnp.int32)
    basef = starts[:, None] + csum - flat
    base3 = (basef.reshape(_NUM_LAYERS, _NW, 16)
             .transpose(1, 0, 2).astype(jnp.int32))
    zs, p = _make_sc_dispatch(_N_TOKENS, _HIDDEN)(z, ids, base3)
    ys = _gmm(zs, W1, b1, W2, b2, step_t, step_g, starts, ends)
    return _make_sc_gather(_N_TOKENS, _HIDDEN)(ys, p)


# speedup vs baseline: 1.6568x; 1.0057x over previous
import functools

import jax
import jax.numpy as jnp
from jax import lax
from jax.experimental import pallas as pl
from jax.experimental.pallas import tpu as pltpu
from jax.experimental.pallas import tpu_sc as plsc

_NUM_LAYERS = 16
_HIDDEN = 768
_INNER = 1536
_N_TOKENS = 32768

_TM = 512
_NTILES = _N_TOKENS // _TM
_NSTEP = _NTILES + _NUM_LAYERS - 1


_NC = 2
_NS = 16
_NW = _NC * _NS


def _make_sc_gather(n_rows, d):
    rpw = n_rows // _NW
    chunk = 64
    nch = rpw // chunk
    mesh = plsc.VectorSubcoreMesh(core_axis_name="c", subcore_axis_name="s")

    @functools.partial(
        pl.kernel,
        mesh=mesh,
        out_type=jax.ShapeDtypeStruct((n_rows, d), jnp.float32),
        scratch_types=[
            pltpu.VMEM((rpw,), jnp.int32),
            pltpu.VMEM((2, chunk, d), jnp.float32),
            pltpu.SemaphoreType.DMA,
            pltpu.SemaphoreType.DMA,
            pltpu.SemaphoreType.DMA,
            pltpu.SemaphoreType.DMA,
        ],
    )
    def gather_k(src_hbm, idx_hbm, out_hbm, idx_v, buf_v, gsem0, gsem1,
                 wsem0, wsem1):
        cid = lax.axis_index("c")
        sid = lax.axis_index("s")
        wid = sid * _NC + cid
        base = wid * rpw
        pltpu.sync_copy(idx_hbm.at[pl.ds(base, rpw)], idx_v)
        gsem = (gsem0, gsem1)
        wsem = (wsem0, wsem1)
        g_cp = [None, None]
        w_cp = [None, None]
        for c in range(nch):
            b = c % 2
            if w_cp[b] is not None:
                w_cp[b].wait()
            g_cp[b] = pltpu.async_copy(
                src_hbm.at[idx_v.at[pl.ds(c * chunk, chunk)]],
                buf_v.at[b], gsem[b])
            if c >= 1:
                pb = (c - 1) % 2
                g_cp[pb].wait()
                w_cp[pb] = pltpu.async_copy(
                    buf_v.at[pb],
                    out_hbm.at[pl.ds(base + (c - 1) * chunk, chunk)],
                    wsem[pb])
        lb = (nch - 1) % 2
        g_cp[lb].wait()
        w_cp[lb] = pltpu.async_copy(
            buf_v.at[lb],
            out_hbm.at[pl.ds(base + (nch - 1) * chunk, chunk)], wsem[lb])
        w_cp[(nch - 2) % 2].wait()
        w_cp[lb].wait()

    return gather_k


_make_sc_gather = functools.lru_cache(maxsize=None)(_make_sc_gather)


def _make_sc_dispatch(n_rows, d):
    rpw = n_rows // _NW
    chunk = 64
    nch = rpw // chunk
    nv = rpw // 16
    mesh = plsc.VectorSubcoreMesh(core_axis_name="c", subcore_axis_name="s")

    @functools.partial(
        pl.kernel,
        mesh=mesh,
        out_type=(
            jax.ShapeDtypeStruct((n_rows, d), jnp.float32),
            jax.ShapeDtypeStruct((n_rows,), jnp.int32),
        ),
        scratch_types=[
            pltpu.VMEM((rpw,), jnp.int32),
            pltpu.VMEM((16, 16), jnp.int32),
            pltpu.VMEM((rpw,), jnp.int32),
            pltpu.VMEM((2, chunk, d), jnp.float32),
            pltpu.SemaphoreType.DMA,
            pltpu.SemaphoreType.DMA,
            pltpu.SemaphoreType.DMA,
            pltpu.SemaphoreType.DMA,
        ],
    )
    def dispatch_k(src_hbm, ids_hbm, base_hbm, zs_hbm, p_hbm,
                   ids_v, base_v, p_v, buf_v, gsem0, gsem1, ssem0, ssem1):
        cid = lax.axis_index("c")
        sid = lax.axis_index("s")
        wid = sid * _NC + cid
        base = wid * rpw
        pltpu.sync_copy(ids_hbm.at[pl.ds(base, rpw)], ids_v)
        pltpu.sync_copy(base_hbm.at[wid], base_v)

        cnt0 = tuple(base_v[l] for l in range(_NUM_LAYERS))

        ones = jnp.full((16,), 1, jnp.int32)
        zeros = jnp.full((16,), 0, jnp.int32)

        def vbody(v, cnts):
            idsv = ids_v[pl.ds(v * 16, 16)]
            slot = zeros
            new = []
            for l in range(_NUM_LAYERS):
                m = idsv == jnp.full((16,), l, jnp.int32)
                slot = jnp.where(m, cnts[l], slot)
                new.append(cnts[l] + jnp.where(m, ones, zeros))
            p_v[pl.ds(v * 16, 16)] = slot
            return tuple(new)

        lax.fori_loop(0, nv, vbody, cnt0)
        pltpu.sync_copy(p_v, p_hbm.at[pl.ds(base, rpw)])

        gsem = (gsem0, gsem1)
        ssem = (ssem0, ssem1)
        g_cp = [None, None]
        s_cp = [[], []]

        def scatter_chunk(c):
            b = c % 2
            for q in range(chunk // 16):
                pv = p_v[pl.ds(c * chunk + q * 16, 16)]
                s_cp[b].append(pltpu.async_copy(
                    buf_v.at[b].at[pl.ds(q * 16, 16)],
                    zs_hbm.at[pv], ssem[b]))

        for c in range(nch):
            b = c % 2
            for cp in s_cp[b]:
                cp.wait()
            s_cp[b] = []
            g_cp[b] = pltpu.async_copy(
                src_hbm.at[pl.ds(base + c * chunk, chunk)], buf_v.at[b],
                gsem[b])
            if c >= 1:
                g_cp[(c - 1) % 2].wait()
                scatter_chunk(c - 1)
        g_cp[(nch - 1) % 2].wait()
        scatter_chunk(nch - 1)
        for b in (0, 1):
            for cp in s_cp[b]:
                cp.wait()

    return dispatch_k


_make_sc_dispatch = functools.lru_cache(maxsize=None)(_make_sc_dispatch)


def _gelu(x):
    return 0.5 * x * (1.0 + lax.erf(x * (2.0 ** -0.5)))


def _gmm_body(st_ref, sg_ref, starts_ref, ends_ref,
              zs_ref, w1_ref, b1_ref, w2_ref, b2_ref, out_ref):
    i = pl.program_id(0)
    t = st_ref[i]
    g = sg_ref[i]
    rows = t * _TM + lax.broadcasted_iota(jnp.int32, (_TM, 1), 0)
    mask = (rows >= starts_ref[g]) & (rows < ends_ref[g])
    x = zs_ref[...]
    h = lax.dot_general(x, w1_ref[0], (((1,), (1,)), ((), ())),
                        preferred_element_type=jnp.float32)
    h = _gelu(h + b1_ref[0])
    y = lax.dot_general(h, w2_ref[0], (((1,), (1,)), ((), ())),
                        preferred_element_type=jnp.float32)
    y = y + b2_ref[0] + x
    out_ref[...] = jnp.where(mask, y, out_ref[...])


def _gmm(zs, W1, b1, W2, b2, step_t, step_g, starts, ends):
    grid_spec = pltpu.PrefetchScalarGridSpec(
        num_scalar_prefetch=4,
        grid=(_NSTEP,),
        in_specs=[
            pl.BlockSpec((_TM, _HIDDEN), lambda i, st, sg, s0, e0: (st[i], 0)),
            pl.BlockSpec((1, _INNER, _HIDDEN),
                         lambda i, st, sg, s0, e0: (sg[i], 0, 0)),
            pl.BlockSpec((1, 1, _INNER), lambda i, st, sg, s0, e0: (sg[i], 0, 0)),
            pl.BlockSpec((1, _HIDDEN, _INNER),
                         lambda i, st, sg, s0, e0: (sg[i], 0, 0)),
            pl.BlockSpec((1, 1, _HIDDEN), lambda i, st, sg, s0, e0: (sg[i], 0, 0)),
        ],
        out_specs=pl.BlockSpec((_TM, _HIDDEN),
                               lambda i, st, sg, s0, e0: (st[i], 0)),
    )
    return pl.pallas_call(
        _gmm_body,
        grid_spec=grid_spec,
        out_shape=jax.ShapeDtypeStruct((_N_TOKENS, _HIDDEN), jnp.float32),
        compiler_params=pltpu.CompilerParams(
            dimension_semantics=("arbitrary",)),
    )(step_t, step_g, starts, ends, zs, W1,
      b1.reshape(_NUM_LAYERS, 1, _INNER), W2,
      b2.reshape(_NUM_LAYERS, 1, _HIDDEN))


def _schedule(sizes):
    ends = jnp.cumsum(sizes)
    starts = ends - sizes
    first = starts // _TM
    count = jnp.where(sizes > 0, (ends - 1) // _TM - first + 1, 0)
    csum = jnp.cumsum(count)
    base = csum - count
    sidx = jnp.arange(_NSTEP, dtype=jnp.int32)
    eff = jnp.minimum(sidx, csum[-1] - 1)
    g = jnp.searchsorted(csum, eff, side="right").astype(jnp.int32)
    t = (first[g] + eff - base[g]).astype(jnp.int32)
    return t, g, starts.astype(jnp.int32), ends.astype(jnp.int32)


def kernel(z, layer_ids, W1, b1, W2, b2):
    ids = layer_ids.astype(jnp.int32)
    rpw = _N_TOKENS // _NW
    i = jnp.arange(_N_TOKENS, dtype=jnp.int32)
    w = i // rpw
    j = i % 16
    hist3 = (jnp.zeros((_NUM_LAYERS, _NW, 16), jnp.int32)
             .at[ids, w, j].add(1, mode="drop"))
    sizes = jnp.sum(hist3, axis=(1, 2))
    step_t, step_g, starts, ends = _schedule(sizes)
    flat = hist3.reshape(_NUM_LAYERS, _NW * 16)
    triu = jnp.triu(jnp.ones((_NW * 16, _NW * 16), jnp.float32))
    csum = jnp.dot(flat.astype(jnp.float32), triu,
                   preferred_element_type=jnp.float32).astype(jnp.int32)
    basef = starts[:, None] + csum - flat
    base3 = (basef.reshape(_NUM_LAYERS, _NW, 16)
             .transpose(1, 0, 2).astype(jnp.int32))
    zs, p = _make_sc_dispatch(_N_TOKENS, _HIDDEN)(z, ids, base3)
    ys = _gmm(zs, W1, b1, W2, b2, step_t, step_g, starts, ends)
    return _make_sc_gather(_N_TOKENS, _HIDDEN)(ys, p)
